# Initial kernel scaffold; baseline (speedup 1.0000x reference)
import jax
import jax.numpy as jnp
from jax.experimental import pallas as pl


def kernel(vi, delta_y, v_previous, x_input, w):
    raise NotImplementedError("write your pallas kernel here")


# speedup vs baseline: 3.3932x; 3.3932x over previous
import functools

import jax
import jax.numpy as jnp
from jax import lax
from jax.experimental import pallas as pl
from jax.experimental.pallas import tpu as pltpu
from jax.experimental.pallas import tpu_sc as plsc

B, T, F = 16384, 200, 14
STEPS = 20
NJ = 4
NC, NS, L = 2, 16, 16
NW = NC * NS
RW = B // NW
RCH = 128
NCH = RW // RCH
NGR = RCH // L

XL_OFF = (T - 2) * F + 12
EARLY_W = STEPS * F


def _sc_phase1(x2, w16):
    mesh = plsc.VectorSubcoreMesh(
        core_axis_name="c", subcore_axis_name="s",
        num_cores=NC, num_subcores=NS)

    @functools.partial(
        pl.kernel,
        out_type=jax.ShapeDtypeStruct((NJ, B, STEPS), jnp.float32),
        mesh=mesh,
        scratch_types=[
            pltpu.VMEM((RCH, EARLY_W), jnp.float32),
            pltpu.VMEM((RCH, 16), jnp.float32),
            pltpu.VMEM((NJ, RCH, STEPS), jnp.float32),
            pltpu.VMEM((L,), jnp.float32),
        ],
        compiler_params=pltpu.CompilerParams(
            use_tc_tiling_on_sc=False, needs_layout_passes=False),
    )
    def k(x_hbm, w_hbm, out_hbm, early_v, xl_v, ahat_v, wv):
        wid = lax.axis_index("s") * NC + lax.axis_index("c")
        pltpu.sync_copy(w_hbm, wv)
        wvec = wv[...]
        lanes = lax.iota(jnp.int32, L)

        def splat_i(v):
            return jnp.full((L,), v, jnp.int32)

        def chunk_body(c, carry):
            base = wid * RW + c * RCH
            pltpu.sync_copy(x_hbm.at[pl.ds(base, RCH), pl.ds(0, EARLY_W)],
                            early_v)
            pltpu.sync_copy(x_hbm.at[pl.ds(base, RCH), pl.ds(XL_OFF, 16)],
                            xl_v)

            def group_body(g, carry2):
                rows = lanes + g * L

                def xl_feat(f):
                    return plsc.load_gather(xl_v, [rows, splat_i(2 + f)])

                d = xl_feat(0)
                dsums = []
                for jj in range(NJ):
                    if jj > 0:
                        d = d + xl_feat(jj)
                    dsums.append(d)
                for jj in range(NJ):
                    denom = wvec + xl_feat(5 + jj) * jnp.float32(25.0)
                    td = dsums[jj] * jnp.float32(150.0) / denom
                    fshift = td * jnp.float32(10.0)
                    tr = fshift.astype(jnp.int32)
                    kk = jnp.where(fshift > tr.astype(jnp.float32),
                                   tr + 1, tr)
                    for i in range(STEPS):
                        idx = jnp.maximum(splat_i(i) - kk, 0)
                        col = idx * F + (10 + jj)
                        val = plsc.load_gather(early_v, [rows, col])
                        plsc.store_scatter(
                            ahat_v, [splat_i(jj), rows, splat_i(i)], val)
                return carry2

            lax.fori_loop(0, NGR, group_body, 0)
            for jj in range(NJ):
                pltpu.sync_copy(ahat_v.at[jj],
                                out_hbm.at[jj, pl.ds(base, RCH)])
            return carry

        lax.fori_loop(0, NCH, chunk_body, 0)

    return k(x2, w16)


def _tc_phase2(ahat):
    BB = 2048
    NB = B // BB

    def body(a0, a1, a2, a3, out_ref, facc):
        refs = (a0, a1, a2, a3)
        p = pl.program_id(0)
        bb = pl.program_id(1)

        @pl.when(p == 0)
        def _():
            for jj in range(NJ):
                a = refs[jj][0]
                m = jnp.max((a != 0.0).astype(jnp.float32),
                            axis=0, keepdims=True)
                prev = jnp.where(bb == 0, jnp.zeros((1, STEPS), jnp.float32),
                                 facc[jj:jj + 1, 0:STEPS])
                facc[jj:jj + 1, 0:STEPS] = jnp.maximum(prev, m)
            out_ref[...] = jnp.zeros_like(out_ref)

        @pl.when(p == 1)
        def _():
            res = jnp.zeros((BB, STEPS), jnp.float32)
            for jj in reversed(range(NJ)):
                fl = facc[jj:jj + 1, 0:STEPS] > 0.0
                res = jnp.where(fl, refs[jj][0], res)
            out_ref[...] = res

    return pl.pallas_call(
        body,
        grid=(2, NB),
        in_specs=[
            pl.BlockSpec((1, BB, STEPS), lambda p, b, jj=jj: (jj, b, 0))
            for jj in range(NJ)
        ],
        out_specs=pl.BlockSpec((BB, STEPS), lambda p, b: (b, 0)),
        out_shape=jax.ShapeDtypeStruct((B, STEPS), jnp.float32),
        scratch_shapes=[pltpu.VMEM((8, 128), jnp.float32)],
    )(ahat, ahat, ahat, ahat)


def kernel(vi, delta_y, v_previous, x_input, w):
    x2 = x_input.reshape(B, T * F)
    w16 = jnp.full((L,), w, jnp.float32)
    ahat = _sc_phase1(x2, w16)
    return _tc_phase2(ahat)


# speedup vs baseline: 4.8463x; 1.4282x over previous
import functools

import jax
import jax.numpy as jnp
from jax import lax
from jax.experimental import pallas as pl
from jax.experimental.pallas import tpu as pltpu
from jax.experimental.pallas import tpu_sc as plsc

B, T, F = 16384, 200, 14
STEPS = 20
NJ = 4
NC, NS, L = 2, 16, 16
NW = NC * NS
RW = B // NW
RCH = 128
NCH = RW // RCH
NGR = RCH // L


def _sc_phase1(xe, xw, w16):
    mesh = plsc.VectorSubcoreMesh(
        core_axis_name="c", subcore_axis_name="s",
        num_cores=NC, num_subcores=NS)

    @functools.partial(
        pl.kernel,
        out_type=jax.ShapeDtypeStruct((NJ, B, STEPS), jnp.float32),
        mesh=mesh,
        scratch_types=[
            pltpu.VMEM((RCH, STEPS, F), jnp.float32),
            pltpu.VMEM((RCH, 1, F), jnp.float32),
            pltpu.VMEM((NJ, RCH, STEPS), jnp.float32),
            pltpu.VMEM((L,), jnp.float32),
        ],
        compiler_params=pltpu.CompilerParams(
            use_tc_tiling_on_sc=False, needs_layout_passes=False),
    )
    def k(xe_hbm, xw_hbm, w_hbm, out_hbm, early_v, xl_v, ahat_v, wv):
        wid = lax.axis_index("s") * NC + lax.axis_index("c")
        pltpu.sync_copy(w_hbm, wv)
        wvec = wv[...]
        lanes = lax.iota(jnp.int32, L)
        zero = jnp.zeros((L,), jnp.int32)

        def splat_i(v):
            return jnp.full((L,), v, jnp.int32)

        def chunk_body(c, carry):
            base = wid * RW + c * RCH
            pltpu.sync_copy(xe_hbm.at[pl.ds(base, RCH)], early_v)
            pltpu.sync_copy(xw_hbm.at[pl.ds(base, RCH)], xl_v)

            def group_body(g, carry2):
                rows = lanes + g * L

                def xl_feat(f):
                    return plsc.load_gather(xl_v, [rows, zero, splat_i(f)])

                d = xl_feat(0)
                dsums = []
                for jj in range(NJ):
                    if jj > 0:
                        d = d + xl_feat(jj)
                    dsums.append(d)
                for jj in range(NJ):
                    denom = wvec + xl_feat(5 + jj) * jnp.float32(25.0)
                    td = dsums[jj] * jnp.float32(150.0) / denom
                    fshift = td * jnp.float32(10.0)
                    tr = fshift.astype(jnp.int32)
                    kk = jnp.where(fshift > tr.astype(jnp.float32),
                                   tr + 1, tr)
                    for i in range(STEPS):
                        idx = jnp.maximum(splat_i(i) - kk, 0)
                        val = plsc.load_gather(
                            early_v, [rows, idx, splat_i(10 + jj)])
                        plsc.store_scatter(
                            ahat_v, [splat_i(jj), rows, splat_i(i)], val)
                return carry2

            lax.fori_loop(0, NGR, group_body, 0)
            for jj in range(NJ):
                pltpu.sync_copy(ahat_v.at[jj],
                                out_hbm.at[jj, pl.ds(base, RCH)])
            return carry

        lax.fori_loop(0, NCH, chunk_body, 0)

    return k(xe, xw, w16)


def _tc_phase2(ahat):
    BB = 2048
    NB = B // BB

    def body(a0, a1, a2, a3, out_ref, facc):
        refs = (a0, a1, a2, a3)
        p = pl.program_id(0)
        bb = pl.program_id(1)

        @pl.when(p == 0)
        def _():
            for jj in range(NJ):
                a = refs[jj][0]
                m = jnp.max((a != 0.0).astype(jnp.float32),
                            axis=0, keepdims=True)
                prev = jnp.where(bb == 0, jnp.zeros((1, STEPS), jnp.float32),
                                 facc[jj:jj + 1, 0:STEPS])
                facc[jj:jj + 1, 0:STEPS] = jnp.maximum(prev, m)
            out_ref[...] = jnp.zeros_like(out_ref)

        @pl.when(p == 1)
        def _():
            res = jnp.zeros((BB, STEPS), jnp.float32)
            for jj in reversed(range(NJ)):
                fl = facc[jj:jj + 1, 0:STEPS] > 0.0
                res = jnp.where(fl, refs[jj][0], res)
            out_ref[...] = res

    return pl.pallas_call(
        body,
        grid=(2, NB),
        in_specs=[
            pl.BlockSpec((1, BB, STEPS), lambda p, b, jj=jj: (jj, b, 0))
            for jj in range(NJ)
        ],
        out_specs=pl.BlockSpec((BB, STEPS), lambda p, b: (b, 0)),
        out_shape=jax.ShapeDtypeStruct((B, STEPS), jnp.float32),
        scratch_shapes=[pltpu.VMEM((8, 128), jnp.float32)],
    )(ahat, ahat, ahat, ahat)


def kernel(vi, delta_y, v_previous, x_input, w):
    xe = x_input[:, 0:STEPS, :]
    xw = x_input[:, T - 1:T, :]
    w16 = jnp.full((L,), w, jnp.float32)
    ahat = _sc_phase1(xe, xw, w16)
    return _tc_phase2(ahat)


# speedup vs baseline: 43.1602x; 8.9059x over previous
import functools

import jax
import jax.numpy as jnp
from jax import lax
from jax.experimental import pallas as pl
from jax.experimental.pallas import tpu as pltpu
from jax.experimental.pallas import tpu_sc as plsc

B, T, F = 16384, 200, 14
STEPS = 20
SROWS = 24
NJ = 4
NC, NS, L = 2, 16, 16
NW = NC * NS
BW = B // NW
BCH = 128
NCH = BW // BCH
NGR = BCH // L


def _sc_phase1(x_t, w16):
    mesh = plsc.VectorSubcoreMesh(
        core_axis_name="c", subcore_axis_name="s",
        num_cores=NC, num_subcores=NS)

    @functools.partial(
        pl.kernel,
        out_type=jax.ShapeDtypeStruct((NJ, SROWS, B), jnp.float32),
        mesh=mesh,
        scratch_types=[
            pltpu.VMEM((NJ, SROWS, BCH), jnp.float32),
            pltpu.VMEM((9, 8, BCH), jnp.float32),
            pltpu.VMEM((NJ, SROWS, BCH), jnp.float32),
            pltpu.VMEM((L,), jnp.float32),
        ],
        compiler_params=pltpu.CompilerParams(
            use_tc_tiling_on_sc=True, needs_layout_passes=False),
    )
    def k(x_hbm, w_hbm, out_hbm, early_v, xl_v, ahat_v, wv):
        wid = lax.axis_index("s") * NC + lax.axis_index("c")
        pltpu.sync_copy(w_hbm, wv)
        wvec = wv[...]
        lanes = lax.iota(jnp.int32, L)

        def splat_i(v):
            return jnp.full((L,), v, jnp.int32)

        def chunk_body(c, carry):
            base = wid * BW + c * BCH
            pltpu.sync_copy(
                x_hbm.at[pl.ds(10, NJ), pl.ds(0, SROWS), pl.ds(base, BCH)],
                early_v)
            pltpu.sync_copy(
                x_hbm.at[pl.ds(0, 9), pl.ds(T - 8, 8), pl.ds(base, BCH)],
                xl_v)

            def group_body(g, carry2):
                sl = pl.ds(g * L, L)
                col = lanes + g * L

                def xl_feat(f):
                    return xl_v[f, 7, sl]

                d = xl_feat(0)
                dsums = []
                for jj in range(NJ):
                    if jj > 0:
                        d = d + xl_feat(jj)
                    dsums.append(d)
                for jj in range(NJ):
                    denom = wvec + xl_feat(5 + jj) * jnp.float32(25.0)
                    td = dsums[jj] * jnp.float32(150.0) / denom
                    fshift = td * jnp.float32(10.0)
                    tr = fshift.astype(jnp.int32)
                    kk = jnp.where(fshift > tr.astype(jnp.float32),
                                   tr + 1, tr)
                    for i in range(STEPS):
                        idx = jnp.maximum(splat_i(i) - kk, 0)
                        val = plsc.load_gather(
                            early_v, [splat_i(jj), idx, col])
                        ahat_v[jj, i, sl] = val
                return carry2

            lax.fori_loop(0, NGR, group_body, 0)
            pltpu.sync_copy(ahat_v,
                            out_hbm.at[:, :, pl.ds(base, BCH)])
            return carry

        lax.fori_loop(0, NCH, chunk_body, 0)

    return k(x_t, w16)


def _tc_phase2(ahat):
    BB = 2048
    NB = B // BB

    def body(a0, a1, a2, a3, out_ref, facc):
        refs = (a0, a1, a2, a3)
        p = pl.program_id(0)
        bb = pl.program_id(1)

        @pl.when(p == 0)
        def _():
            for jj in range(NJ):
                a = refs[jj][0]
                m = jnp.max((a[0:STEPS, :] != 0.0).astype(jnp.float32),
                            axis=1, keepdims=True)
                prev = jnp.where(bb == 0, jnp.zeros((STEPS, 1), jnp.float32),
                                 facc[0:STEPS, jj:jj + 1])
                facc[0:STEPS, jj:jj + 1] = jnp.maximum(prev, m)
            out_ref[...] = jnp.zeros_like(out_ref)

        @pl.when(p == 1)
        def _():
            res = jnp.zeros((STEPS, BB), jnp.float32)
            for jj in reversed(range(NJ)):
                fl = facc[0:STEPS, jj:jj + 1] > 0.0
                res = jnp.where(fl, refs[jj][0][0:STEPS, :], res)
            out_ref[...] = res

    return pl.pallas_call(
        body,
        grid=(2, NB),
        in_specs=[
            pl.BlockSpec((1, SROWS, BB), lambda p, b, jj=jj: (jj, 0, b))
            for jj in range(NJ)
        ],
        out_specs=pl.BlockSpec((STEPS, BB), lambda p, b: (0, b)),
        out_shape=jax.ShapeDtypeStruct((STEPS, B), jnp.float32),
        scratch_shapes=[pltpu.VMEM((SROWS, 128), jnp.float32)],
    )(ahat, ahat, ahat, ahat)


def kernel(vi, delta_y, v_previous, x_input, w):
    x_t = jnp.transpose(x_input, (2, 1, 0))
    w16 = jnp.full((L,), w, jnp.float32)
    ahat = _sc_phase1(x_t, w16)
    out_t = _tc_phase2(ahat)
    return out_t.T


# speedup vs baseline: 50.5896x; 1.1721x over previous
import functools

import jax
import jax.numpy as jnp
from jax import lax
from jax.experimental import pallas as pl
from jax.experimental.pallas import tpu as pltpu
from jax.experimental.pallas import tpu_sc as plsc

B, T, F = 16384, 200, 14
STEPS = 20
SROWS = 24
NJ = 4
NC, NS, L = 2, 16, 16
NW = NC * NS
BW = B // NW
BCH = 256
NCH = BW // BCH
NGR = BCH // L


def _sc_phase1(x_t, w16):
    mesh = plsc.VectorSubcoreMesh(
        core_axis_name="c", subcore_axis_name="s",
        num_cores=NC, num_subcores=NS)

    @functools.partial(
        pl.kernel,
        out_type=jax.ShapeDtypeStruct((NJ, STEPS, B), jnp.float32),
        mesh=mesh,
        scratch_types=[
            pltpu.VMEM((NJ, SROWS, BCH), jnp.float32),
            pltpu.VMEM((NJ, SROWS, BCH), jnp.float32),
            pltpu.VMEM((9, 8, BCH), jnp.float32),
            pltpu.VMEM((9, 8, BCH), jnp.float32),
            pltpu.VMEM((NJ, STEPS, BCH), jnp.float32),
            pltpu.VMEM((L,), jnp.float32),
            pltpu.SemaphoreType.DMA,
            pltpu.SemaphoreType.DMA,
            pltpu.SemaphoreType.DMA,
            pltpu.SemaphoreType.DMA,
            pltpu.SemaphoreType.DMA,
            pltpu.SemaphoreType.DMA,
        ],
        compiler_params=pltpu.CompilerParams(
            use_tc_tiling_on_sc=True, needs_layout_passes=False),
    )
    def k(x_hbm, w_hbm, out_hbm, e0, e1, x0, x1, av, wv,
          se0, se1, sx0, sx1, so0, so1):
        wid = lax.axis_index("s") * NC + lax.axis_index("c")
        base0 = wid * BW
        base1 = base0 + BCH

        def in_copies(base, ev, xv, sem_e, sem_x):
            ce = pltpu.async_copy(
                x_hbm.at[pl.ds(10, NJ), pl.ds(0, SROWS), pl.ds(base, BCH)],
                ev, sem_e)
            cx = pltpu.async_copy(
                x_hbm.at[pl.ds(0, 9), pl.ds(T - 8, 8), pl.ds(base, BCH)],
                xv, sem_x)
            return ce, cx

        ce0, cx0 = in_copies(base0, e0, x0, se0, sx0)
        ce1, cx1 = in_copies(base1, e1, x1, se1, sx1)
        pltpu.sync_copy(w_hbm, wv)
        wvec = wv[...]
        lanes = lax.iota(jnp.int32, L)

        def splat_i(v):
            return jnp.full((L,), v, jnp.int32)

        def compute(early_v, xl_v, ahat_v):
            def group_body(g, carry2):
                sl = pl.ds(g * L, L)
                col = lanes + g * L

                def xl_feat(f):
                    return xl_v[f, 7, sl]

                d = xl_feat(0)
                dsums = []
                for jj in range(NJ):
                    if jj > 0:
                        d = d + xl_feat(jj)
                    dsums.append(d)
                for jj in range(NJ):
                    denom = wvec + xl_feat(5 + jj) * jnp.float32(25.0)
                    td = dsums[jj] * jnp.float32(150.0) / denom
                    fshift = td * jnp.float32(10.0)
                    tr = fshift.astype(jnp.int32)
                    kk = jnp.where(fshift > tr.astype(jnp.float32),
                                   tr + 1, tr)
                    for i in range(STEPS):
                        idx = jnp.maximum(splat_i(i) - kk, 0)
                        val = plsc.load_gather(
                            early_v, [splat_i(jj), idx, col])
                        ahat_v[jj, i, sl] = val
                return carry2

            lax.fori_loop(0, NGR, group_body, 0)

        ce0.wait()
        cx0.wait()
        compute(e0, x0, av)
        co0 = pltpu.async_copy(av, out_hbm.at[:, :, pl.ds(base0, BCH)], so0)
        ce1.wait()
        cx1.wait()
        co0.wait()
        compute(e1, x1, av)
        co1 = pltpu.async_copy(av, out_hbm.at[:, :, pl.ds(base1, BCH)], so1)
        co1.wait()

    return k(x_t, w16)


def _tc_phase2(ahat):
    BB = 4096
    NB = B // BB

    def body(a0, a1, a2, a3, out_ref, facc):
        refs = (a0, a1, a2, a3)
        p = pl.program_id(0)
        bb = pl.program_id(1)

        @pl.when(p == 0)
        def _():
            for jj in range(NJ):
                a = refs[jj][0]
                m = jnp.max((a != 0.0).astype(jnp.float32),
                            axis=1, keepdims=True)
                prev = jnp.where(bb == 0, jnp.zeros((STEPS, 1), jnp.float32),
                                 facc[0:STEPS, jj:jj + 1])
                facc[0:STEPS, jj:jj + 1] = jnp.maximum(prev, m)
            out_ref[...] = jnp.zeros_like(out_ref)

        @pl.when(p == 1)
        def _():
            res = jnp.zeros((STEPS, BB), jnp.float32)
            for jj in reversed(range(NJ)):
                fl = facc[0:STEPS, jj:jj + 1] > 0.0
                res = jnp.where(fl, refs[jj][0], res)
            out_ref[...] = res

    return pl.pallas_call(
        body,
        grid=(2, NB),
        in_specs=[
            pl.BlockSpec((1, STEPS, BB), lambda p, b, jj=jj: (jj, 0, b))
            for jj in range(NJ)
        ],
        out_specs=pl.BlockSpec((STEPS, BB), lambda p, b: (0, b)),
        out_shape=jax.ShapeDtypeStruct((STEPS, B), jnp.float32),
        scratch_shapes=[pltpu.VMEM((SROWS, 128), jnp.float32)],
    )(ahat, ahat, ahat, ahat)


def kernel(vi, delta_y, v_previous, x_input, w):
    x_t = jnp.transpose(x_input, (2, 1, 0))
    w16 = jnp.full((L,), w, jnp.float32)
    ahat = _sc_phase1(x_t, w16)
    out_t = _tc_phase2(ahat)
    return out_t.T
